# baseline (speedup 1.0000x reference)
import functools

import jax
import jax.numpy as jnp
from jax import lax
from jax.experimental import pallas as pl
from jax.experimental.pallas import tpu as pltpu
from jax.experimental.pallas import tpu_sc as plsc

R = 128
C = 100000
W = 1024
NBLK1 = 90
CSPLIT = NBLK1 * W
SCW = 8192
NBLK2 = SCW // W
ROWS_PER_SUBCORE = 4
LANES = 16

_ROT0 = (13, 15, 26, 6)
_ROT1 = (17, 29, 16, 24)
_TINY = float(jnp.finfo(jnp.float32).tiny)


def _rotl(x, r):
    return (x << jnp.uint32(r)) | (x >> jnp.uint32(32 - r))


def _threefry_bits(x1):
    ks0 = jnp.uint32(0)
    ks1 = jnp.uint32(42)
    ks2 = ks0 ^ ks1 ^ jnp.uint32(0x1BD11BDA)
    ks = (ks0, ks1, ks2)
    x0 = jnp.zeros_like(x1)
    for g in range(5):
        rots = _ROT0 if g % 2 == 0 else _ROT1
        for r in rots:
            x0 = x0 + x1
            x1 = _rotl(x1, r)
            x1 = x1 ^ x0
        x0 = x0 + ks[(g + 1) % 3]
        x1 = x1 + ks[(g + 2) % 3] + jnp.uint32(g + 1)
    return x0 ^ x1


def _bits_to_u(bits):
    fb = (bits >> jnp.uint32(9)) | jnp.uint32(0x3F800000)
    f = lax.bitcast_convert_type(fb, jnp.float32) - jnp.float32(1.0)
    return jnp.maximum(f, jnp.float32(_TINY))


def _sc_uniform_body(u_hbm, buf, sem):
    wid = lax.axis_index("s") * 2 + lax.axis_index("c")
    row0 = wid * ROWS_PER_SUBCORE
    iota = lax.iota(jnp.uint32, LANES)

    def step(i, carry):
        coff = jnp.uint32(CSPLIT + 42) + jnp.uint32(LANES) * i.astype(jnp.uint32)
        for r in range(ROWS_PER_SUBCORE):
            base = (row0 + r).astype(jnp.uint32) * jnp.uint32(C) + coff
            x1 = base + iota
            u = _bits_to_u(_threefry_bits(x1))
            buf[r, pl.ds(i * LANES, LANES)] = u
        return carry

    lax.fori_loop(0, SCW // LANES, step, jnp.int32(0))
    pltpu.async_copy(buf, u_hbm.at[pl.ds(row0, ROWS_PER_SUBCORE), :], sem).wait()


def _sc_uniform():
    return pl.kernel(
        _sc_uniform_body,
        mesh=plsc.VectorSubcoreMesh(core_axis_name="c", subcore_axis_name="s"),
        out_type=jax.ShapeDtypeStruct((R, SCW), jnp.float32),
        scratch_types=[
            pltpu.VMEM((ROWS_PER_SUBCORE, SCW), jnp.float32),
            pltpu.SemaphoreType.DMA,
        ],
    )()


def _main_body(x_ref, val_ref, idx_ref, valc_ref, idxc_ref, x1c_ref):
    b = pl.program_id(0)

    @pl.when(b == 0)
    def _init():
        valc_ref[...] = jnp.full((R, 1), -1.0, jnp.float32)
        idxc_ref[...] = jnp.zeros((R, 1), jnp.int32)
        row = jax.lax.broadcasted_iota(jnp.uint32, (R, W), 0)
        col0 = jax.lax.broadcasted_iota(jnp.uint32, (R, W), 1)
        x1c_ref[...] = row * jnp.uint32(C) + col0 + jnp.uint32(42)

    x = x_ref[...]
    m = jnp.max(x, axis=0, keepdims=True)
    e = jnp.exp(x - m)
    s = jax.lax.dot_general(
        jnp.ones((1, R), jnp.float32), e,
        dimension_numbers=(((1,), (0,)), ((), ())),
        preferred_element_type=jnp.float32)

    x1 = x1c_ref[...]
    x1c_ref[...] = x1 + jnp.uint32(W)
    u = _bits_to_u(_threefry_bits(x1))
    t = -jnp.log(u)

    score = e / (s * t)

    bm = jnp.max(score, axis=1, keepdims=True)
    is_max = score == bm
    lidx = jnp.min(
        jnp.where(is_max, jax.lax.broadcasted_iota(jnp.int32, (R, W), 1),
                  jnp.int32(0x7FFFFFFF)),
        axis=1, keepdims=True)
    cand_idx = lidx + jnp.int32(W) * b

    better = bm > valc_ref[...]
    valc_ref[...] = jnp.where(better, bm, valc_ref[...])
    idxc_ref[...] = jnp.where(better, cand_idx, idxc_ref[...])

    @pl.when(b == NBLK1 - 1)
    def _emit():
        val_ref[...] = valc_ref[...]
        idx_ref[...] = idxc_ref[...]


def _tail_body(x_ref, u_ref, val_in, idx_in, o_ref, valc_ref, idxc_ref):
    b = pl.program_id(0)

    @pl.when(b == 0)
    def _init():
        valc_ref[...] = val_in[...]
        idxc_ref[...] = idx_in[...]

    x = x_ref[...]
    m = jnp.max(x, axis=0, keepdims=True)
    e = jnp.exp(x - m)
    s = jax.lax.dot_general(
        jnp.ones((1, R), jnp.float32), e,
        dimension_numbers=(((1,), (0,)), ((), ())),
        preferred_element_type=jnp.float32)

    t = -jnp.log(u_ref[...])

    score = e / (s * t)
    col = jax.lax.broadcasted_iota(jnp.int32, (1, W), 1)
    valid = col < (jnp.int32(C - CSPLIT) - jnp.int32(W) * b)
    score = jnp.where(valid, score, jnp.float32(-1.0))

    bm = jnp.max(score, axis=1, keepdims=True)
    is_max = score == bm
    lidx = jnp.min(
        jnp.where(is_max, jax.lax.broadcasted_iota(jnp.int32, (R, W), 1),
                  jnp.int32(0x7FFFFFFF)),
        axis=1, keepdims=True)
    cand_idx = lidx + (jnp.int32(CSPLIT) + jnp.int32(W) * b)

    better = bm > valc_ref[...]
    valc_ref[...] = jnp.where(better, bm, valc_ref[...])
    idxc_ref[...] = jnp.where(better, cand_idx, idxc_ref[...])

    @pl.when(b == NBLK2 - 1)
    def _emit():
        o_ref[...] = idxc_ref[...]


@jax.jit
def kernel(outputs):
    u = _sc_uniform()
    val, idx = pl.pallas_call(
        _main_body,
        grid=(NBLK1,),
        in_specs=[pl.BlockSpec((R, W), lambda b: (0, b))],
        out_specs=[
            pl.BlockSpec((R, 1), lambda b: (0, 0)),
            pl.BlockSpec((R, 1), lambda b: (0, 0)),
        ],
        out_shape=[
            jax.ShapeDtypeStruct((R, 1), jnp.float32),
            jax.ShapeDtypeStruct((R, 1), jnp.int32),
        ],
        scratch_shapes=[
            pltpu.VMEM((R, 1), jnp.float32),
            pltpu.VMEM((R, 1), jnp.int32),
            pltpu.VMEM((R, W), jnp.uint32),
        ],
    )(outputs)
    return pl.pallas_call(
        _tail_body,
        grid=(NBLK2,),
        in_specs=[
            pl.BlockSpec((R, W), lambda b: (0, NBLK1 + b)),
            pl.BlockSpec((R, W), lambda b: (0, b)),
            pl.BlockSpec((R, 1), lambda b: (0, 0)),
            pl.BlockSpec((R, 1), lambda b: (0, 0)),
        ],
        out_specs=pl.BlockSpec((R, 1), lambda b: (0, 0)),
        out_shape=jax.ShapeDtypeStruct((R, 1), jnp.int32),
        scratch_shapes=[
            pltpu.VMEM((R, 1), jnp.float32),
            pltpu.VMEM((R, 1), jnp.int32),
        ],
    )(outputs, u, val, idx)


# speedup vs baseline: 1.1466x; 1.1466x over previous
import functools

import jax
import jax.numpy as jnp
from jax import lax
from jax.experimental import pallas as pl
from jax.experimental.pallas import tpu as pltpu
from jax.experimental.pallas import tpu_sc as plsc

R = 128
C = 100000
W = 1024
NBLK1 = 66
CSPLIT = NBLK1 * W
SCW = 32768
NBLK2 = SCW // W
ROWS_PER_SUBCORE = 4
LANES = 16
SC_CHUNK = 8192
NCHUNK = SCW // SC_CHUNK

_ROT0 = (13, 15, 26, 6)
_ROT1 = (17, 29, 16, 24)
_TINY = float(jnp.finfo(jnp.float32).tiny)


def _rotl(x, r):
    return (x << jnp.uint32(r)) | (x >> jnp.uint32(32 - r))


def _threefry_bits(x1):
    ks0 = jnp.uint32(0)
    ks1 = jnp.uint32(42)
    ks2 = ks0 ^ ks1 ^ jnp.uint32(0x1BD11BDA)
    ks = (ks0, ks1, ks2)
    x0 = jnp.zeros_like(x1)
    for g in range(5):
        rots = _ROT0 if g % 2 == 0 else _ROT1
        for r in rots:
            x0 = x0 + x1
            x1 = _rotl(x1, r)
            x1 = x1 ^ x0
        x0 = x0 + ks[(g + 1) % 3]
        x1 = x1 + ks[(g + 2) % 3] + jnp.uint32(g + 1)
    return x0 ^ x1


def _bits_to_u(bits):
    fb = (bits >> jnp.uint32(9)) | jnp.uint32(0x3F800000)
    f = lax.bitcast_convert_type(fb, jnp.float32) - jnp.float32(1.0)
    return jnp.maximum(f, jnp.float32(_TINY))


def _sc_uniform_body(u_hbm, buf0, buf1, sem0, sem1):
    wid = lax.axis_index("s") * 2 + lax.axis_index("c")
    row0 = wid * ROWS_PER_SUBCORE
    iota = lax.iota(jnp.uint32, LANES)
    bufs = (buf0, buf1)
    sems = (sem0, sem1)
    pending = [None, None]

    for chunk in range(NCHUNK):
        par = chunk & 1
        if pending[par] is not None:
            pending[par].wait()
        buf = bufs[par]

        def step(i, carry, _chunk=chunk, _buf=buf):
            coff = (jnp.uint32(CSPLIT + 42 + _chunk * SC_CHUNK)
                    + jnp.uint32(LANES) * i.astype(jnp.uint32))
            for r in range(ROWS_PER_SUBCORE):
                base = (row0 + r).astype(jnp.uint32) * jnp.uint32(C) + coff
                x1 = base + iota
                u = _bits_to_u(_threefry_bits(x1))
                _buf[r, pl.ds(i * LANES, LANES)] = u
            return carry

        lax.fori_loop(0, SC_CHUNK // LANES, step, jnp.int32(0))
        pending[par] = pltpu.async_copy(
            buf,
            u_hbm.at[pl.ds(row0, ROWS_PER_SUBCORE),
                     pl.ds(chunk * SC_CHUNK, SC_CHUNK)],
            sems[par])

    for p in pending:
        if p is not None:
            p.wait()


def _sc_uniform():
    return pl.kernel(
        _sc_uniform_body,
        mesh=plsc.VectorSubcoreMesh(core_axis_name="c", subcore_axis_name="s"),
        out_type=jax.ShapeDtypeStruct((R, SCW), jnp.float32),
        scratch_types=[
            pltpu.VMEM((ROWS_PER_SUBCORE, SC_CHUNK), jnp.float32),
            pltpu.VMEM((ROWS_PER_SUBCORE, SC_CHUNK), jnp.float32),
            pltpu.SemaphoreType.DMA,
            pltpu.SemaphoreType.DMA,
        ],
    )()


def _main_body(x_ref, val_ref, idx_ref, valc_ref, idxc_ref, x1c_ref):
    b = pl.program_id(0)

    @pl.when(b == 0)
    def _init():
        valc_ref[...] = jnp.full((R, 1), -1.0, jnp.float32)
        idxc_ref[...] = jnp.zeros((R, 1), jnp.int32)
        row = jax.lax.broadcasted_iota(jnp.uint32, (R, W), 0)
        col0 = jax.lax.broadcasted_iota(jnp.uint32, (R, W), 1)
        x1c_ref[...] = row * jnp.uint32(C) + col0 + jnp.uint32(42)

    x = x_ref[...]
    m = jnp.max(x, axis=0, keepdims=True)
    e = jnp.exp(x - m)
    s = jax.lax.dot_general(
        jnp.ones((1, R), jnp.float32), e,
        dimension_numbers=(((1,), (0,)), ((), ())),
        preferred_element_type=jnp.float32)

    x1 = x1c_ref[...]
    x1c_ref[...] = x1 + jnp.uint32(W)
    u = _bits_to_u(_threefry_bits(x1))
    t = -jnp.log(u)

    score = e / (s * t)

    bm = jnp.max(score, axis=1, keepdims=True)
    is_max = score == bm
    lidx = jnp.min(
        jnp.where(is_max, jax.lax.broadcasted_iota(jnp.int32, (R, W), 1),
                  jnp.int32(0x7FFFFFFF)),
        axis=1, keepdims=True)
    cand_idx = lidx + jnp.int32(W) * b

    better = bm > valc_ref[...]
    valc_ref[...] = jnp.where(better, bm, valc_ref[...])
    idxc_ref[...] = jnp.where(better, cand_idx, idxc_ref[...])

    @pl.when(b == NBLK1 - 1)
    def _emit():
        val_ref[...] = valc_ref[...]
        idx_ref[...] = idxc_ref[...]


def _tail_body(x_ref, u_ref, val_in, idx_in, o_ref, valc_ref, idxc_ref):
    b = pl.program_id(0)

    @pl.when(b == 0)
    def _init():
        valc_ref[...] = val_in[...]
        idxc_ref[...] = idx_in[...]

    x = x_ref[...]
    m = jnp.max(x, axis=0, keepdims=True)
    e = jnp.exp(x - m)
    s = jax.lax.dot_general(
        jnp.ones((1, R), jnp.float32), e,
        dimension_numbers=(((1,), (0,)), ((), ())),
        preferred_element_type=jnp.float32)

    t = -jnp.log(u_ref[...])

    score = e / (s * t)
    col = jax.lax.broadcasted_iota(jnp.int32, (1, W), 1)
    valid = col < (jnp.int32(C - CSPLIT) - jnp.int32(W) * b)
    score = jnp.where(valid, score, jnp.float32(-1.0))

    bm = jnp.max(score, axis=1, keepdims=True)
    is_max = score == bm
    lidx = jnp.min(
        jnp.where(is_max, jax.lax.broadcasted_iota(jnp.int32, (R, W), 1),
                  jnp.int32(0x7FFFFFFF)),
        axis=1, keepdims=True)
    cand_idx = lidx + (jnp.int32(CSPLIT) + jnp.int32(W) * b)

    better = bm > valc_ref[...]
    valc_ref[...] = jnp.where(better, bm, valc_ref[...])
    idxc_ref[...] = jnp.where(better, cand_idx, idxc_ref[...])

    @pl.when(b == NBLK2 - 1)
    def _emit():
        o_ref[...] = idxc_ref[...]


@jax.jit
def kernel(outputs):
    u = _sc_uniform()
    val, idx = pl.pallas_call(
        _main_body,
        grid=(NBLK1,),
        in_specs=[pl.BlockSpec((R, W), lambda b: (0, b))],
        out_specs=[
            pl.BlockSpec((R, 1), lambda b: (0, 0)),
            pl.BlockSpec((R, 1), lambda b: (0, 0)),
        ],
        out_shape=[
            jax.ShapeDtypeStruct((R, 1), jnp.float32),
            jax.ShapeDtypeStruct((R, 1), jnp.int32),
        ],
        scratch_shapes=[
            pltpu.VMEM((R, 1), jnp.float32),
            pltpu.VMEM((R, 1), jnp.int32),
            pltpu.VMEM((R, W), jnp.uint32),
        ],
    )(outputs)
    return pl.pallas_call(
        _tail_body,
        grid=(NBLK2,),
        in_specs=[
            pl.BlockSpec((R, W), lambda b: (0, NBLK1 + b)),
            pl.BlockSpec((R, W), lambda b: (0, b)),
            pl.BlockSpec((R, 1), lambda b: (0, 0)),
            pl.BlockSpec((R, 1), lambda b: (0, 0)),
        ],
        out_specs=pl.BlockSpec((R, 1), lambda b: (0, 0)),
        out_shape=jax.ShapeDtypeStruct((R, 1), jnp.int32),
        scratch_shapes=[
            pltpu.VMEM((R, 1), jnp.float32),
            pltpu.VMEM((R, 1), jnp.int32),
        ],
    )(outputs, u, val, idx)
